# baseline (speedup 1.0000x reference)
import jax
import jax.numpy as jnp
from jax.experimental import pallas as pl
from jax.experimental.pallas import tpu as pltpu

N = 100000
NPAD = 100352


def _final_mlp_body(x_ref, wh1_ref, bh1_ref, wh2_ref, bh2_ref, o_ref):
    h = jnp.maximum(x_ref[...] @ wh1_ref[...] + bh1_ref[...], 0.0)
    z = h @ wh2_ref[...] + bh2_ref[...]
    o_ref[...] = 1.0 / (1.0 + jnp.exp(-z))


def kernel(x, edge_index, edge_attr, W1a, b1a, W1b, b1b, W2a, b2a, W2b, b2b, Wh1, bh1, Wh2, bh2):
    src = edge_index[0]
    dst = edge_index[1]

    def conv(xin):
        x_j = jnp.take(xin, src, axis=0)
        tmp = jnp.concatenate([x_j, edge_attr], axis=1)
        h = jax.nn.relu(tmp @ W1a + b1a)
        msg = jax.nn.relu(h @ W1b + b1b)
        agg = jax.ops.segment_max(msg, dst, num_segments=xin.shape[0])
        agg = jnp.where(jnp.isfinite(agg), agg, 0.0)
        cat = jnp.concatenate([xin, agg], axis=1)
        comb = jax.nn.relu(jax.nn.relu(cat @ W2a + b2a) @ W2b + b2b)
        return jnp.concatenate([xin[:, :1], comb], axis=1)

    x1 = conv(x)
    x2 = conv(x1)
    out = conv(x2)

    feat = out[:, 1:]
    feat = jnp.pad(feat, ((0, NPAD - N), (0, 0)))
    blk = 12544
    res = pl.pallas_call(
        _final_mlp_body,
        grid=(NPAD // blk,),
        in_specs=[
            pl.BlockSpec((blk, 8), lambda i: (i, 0)),
            pl.BlockSpec((8, 16), lambda i: (0, 0)),
            pl.BlockSpec((16,), lambda i: (0,)),
            pl.BlockSpec((16, 1), lambda i: (0, 0)),
            pl.BlockSpec((1,), lambda i: (0,)),
        ],
        out_specs=pl.BlockSpec((blk, 1), lambda i: (i, 0)),
        out_shape=jax.ShapeDtypeStruct((NPAD, 1), jnp.float32),
    )(feat, Wh1, bh1, Wh2, bh2)
    return res[:N]


# speedup vs baseline: 2.9854x; 2.9854x over previous
import functools

import jax
import jax.numpy as jnp
from jax import lax
from jax.experimental import pallas as pl
from jax.experimental.pallas import tpu as pltpu
from jax.experimental.pallas import tpu_sc as plsc

N = 100000
E = 3200000
NB = 64
NPB = 1600
NAGG = NB * NPB
EPW = E // 32
CHB = 504
CHD = 256
KPAD = E + CHD
PMASK = (1 << 22) - 1
BE = 4000
BN = 4000

_SC_MESH = plsc.VectorSubcoreMesh(core_axis_name="c", subcore_axis_name="s")


def _gather_body(xpad_hbm, src_hbm, g_hbm, idx_v, gbuf, sem):
    c = lax.axis_index("c")
    s = lax.axis_index("s")
    base = (s * 2 + c) * EPW

    nfull = EPW // CHB

    def chunk(k, _):
        off = base + jnp.minimum(k * CHB, EPW - CHB)
        pltpu.sync_copy(src_hbm.at[pl.ds(off, CHB)], idx_v)
        pltpu.async_copy(xpad_hbm.at[idx_v], gbuf, sem).wait()
        pltpu.sync_copy(gbuf, g_hbm.at[pl.ds(off, CHB)])
        return 0

    lax.fori_loop(0, nfull + (1 if EPW % CHB else 0), chunk, 0)


_gather = pl.kernel(
    _gather_body,
    out_type=jax.ShapeDtypeStruct((E, 16), jnp.float32),
    mesh=_SC_MESH,
    compiler_params=pltpu.CompilerParams(use_tc_tiling_on_sc=False),
    scratch_types=[
        pltpu.VMEM((CHB,), jnp.int32),
        pltpu.VMEM((CHB, 16), jnp.float32),
        pltpu.SemaphoreType.DMA,
    ],
)


def _scatter_body(msg_hbm, key_hbm, dst_hbm, lim_hbm, agg_hbm,
                  idx_v, mbuf, dbuf, kbuf, lim_s, acc, sem):
    c = lax.axis_index("c")
    s = lax.axis_index("s")
    tile = s * 2 + c

    pltpu.sync_copy(lim_hbm, lim_s)

    def one_bucket(w):
        def initrow(i, _):
            acc[pl.ds(i * 16, 16)] = jnp.full((16,), -jnp.inf, jnp.float32)
            return 0

        lax.fori_loop(0, (NPB + 1) * 2, initrow, 0)

        lv = lim_s[pl.ds(w, 16)]
        start = lv[0]
        end = lv[1]
        start_al = (start // CHD) * CHD
        nchunks = (end - start_al + CHD - 1) // CHD
        nbase = w * NPB

        def chunk(k, _):
            off = start_al + k * CHD
            pltpu.sync_copy(key_hbm.at[pl.ds(off, CHD)], kbuf)

            def mkidx(vi, _):
                b = vi * 16
                idx_v[pl.ds(b, 16)] = kbuf[pl.ds(b, 16)] & PMASK
                return 0

            lax.fori_loop(0, CHD // 16, mkidx, 0)
            cp = pltpu.async_copy(msg_hbm.at[idx_v], mbuf, sem)
            pltpu.sync_copy(dst_hbm.at[pl.ds(off, CHD)], dbuf)
            cp.wait()
            lo = start - off
            hi = end - off

            def vec16(vi, _):
                base = vi * 16
                lane = lax.iota(jnp.int32, 16) + base
                inb = (lane >= lo) & (lane < hi)
                dv = jnp.where(inb, dbuf[pl.ds(base, 16)] - nbase, NPB)
                for j in range(16):
                    a = dv[j] * 32
                    i = base + j
                    acc[pl.ds(a, 16)] = jnp.maximum(acc[pl.ds(a, 16)], mbuf[i, 0:16])
                    acc[pl.ds(a + 16, 16)] = jnp.maximum(
                        acc[pl.ds(a + 16, 16)], mbuf[i, 16:32])
                return 0

            lax.fori_loop(0, CHD // 16, vec16, 0)
            return 0

        lax.fori_loop(0, nchunks, chunk, 0)
        pltpu.sync_copy(acc.at[pl.ds(0, NPB * 32)],
                        agg_hbm.at[pl.ds(nbase * 32, NPB * 32)])

    one_bucket(tile * 2)
    one_bucket(tile * 2 + 1)


_scatter = pl.kernel(
    _scatter_body,
    out_type=jax.ShapeDtypeStruct((NAGG * 32,), jnp.float32),
    mesh=_SC_MESH,
    compiler_params=pltpu.CompilerParams(use_tc_tiling_on_sc=False),
    scratch_types=[
        pltpu.VMEM((CHD,), jnp.int32),
        pltpu.VMEM((CHD, 32), jnp.float32),
        pltpu.VMEM((CHD,), jnp.int32),
        pltpu.VMEM((CHD,), jnp.int32),
        pltpu.VMEM((80,), jnp.int32),
        pltpu.VMEM(((NPB + 1) * 32,), jnp.float32),
        pltpu.SemaphoreType.DMA,
    ],
)


def _edge_mlp_body(g_ref, a_ref, w1_ref, w10_ref, b1_ref, w2_ref, b2_ref, o_ref):
    g = g_ref[...]
    t = a_ref[...]
    h = jnp.maximum(g @ w1_ref[...] + t * w10_ref[...] + b1_ref[...], 0.0)
    o_ref[...] = jnp.maximum(h @ w2_ref[...] + b2_ref[...], 0.0)


def _edge_mlp(g, attr, w1p, w10, b1, w2, b2):
    return pl.pallas_call(
        _edge_mlp_body,
        grid=(E // BE,),
        in_specs=[
            pl.BlockSpec((BE, 16), lambda i: (i, 0)),
            pl.BlockSpec((BE, 1), lambda i: (i, 0)),
            pl.BlockSpec((16, 32), lambda i: (0, 0)),
            pl.BlockSpec((1, 32), lambda i: (0, 0)),
            pl.BlockSpec((1, 32), lambda i: (0, 0)),
            pl.BlockSpec((32, 32), lambda i: (0, 0)),
            pl.BlockSpec((1, 32), lambda i: (0, 0)),
        ],
        out_specs=pl.BlockSpec((BE, 32), lambda i: (i, 0)),
        out_shape=jax.ShapeDtypeStruct((E, 32), jnp.float32),
    )(g, attr, w1p, w10, b1, w2, b2)


def _node_mlp_body(final, xp_ref, agg_ref, w2ax_ref, w2ag_ref, b2a_ref,
                   w2b_ref, b2b_ref, q_ref, p_ref, wh1_ref, bh1_ref,
                   wh2_ref, bh2_ref, o_ref):
    xp = xp_ref[...]
    agg = agg_ref[...]
    agg = jnp.where(agg == -jnp.inf, 0.0, agg)
    h = jnp.maximum(xp @ w2ax_ref[...] + agg @ w2ag_ref[...] + b2a_ref[...], 0.0)
    comb = jnp.maximum(h @ w2b_ref[...] + b2b_ref[...], 0.0)
    if final:
        h2 = jnp.maximum(comb @ wh1_ref[...] + bh1_ref[...], 0.0)
        z = h2 @ wh2_ref[...] + bh2_ref[...]
        o_ref[...] = 1.0 / (1.0 + jnp.exp(-z))
    else:
        o_ref[...] = xp @ q_ref[...] + comb @ p_ref[...]


def _node_mlp(final, xp, agg, w2ax, w2ag, b2a, w2b, b2b, q, p, wh1, bh1, wh2, bh2):
    out_cols = 1 if final else 16
    return pl.pallas_call(
        functools.partial(_node_mlp_body, final),
        grid=(N // BN,),
        in_specs=[
            pl.BlockSpec((BN, 16), lambda i: (i, 0)),
            pl.BlockSpec((BN, 32), lambda i: (i, 0)),
            pl.BlockSpec((16, 16), lambda i: (0, 0)),
            pl.BlockSpec((32, 16), lambda i: (0, 0)),
            pl.BlockSpec((1, 16), lambda i: (0, 0)),
            pl.BlockSpec((16, 8), lambda i: (0, 0)),
            pl.BlockSpec((1, 8), lambda i: (0, 0)),
            pl.BlockSpec((16, 16), lambda i: (0, 0)),
            pl.BlockSpec((8, 16), lambda i: (0, 0)),
            pl.BlockSpec((8, 16), lambda i: (0, 0)),
            pl.BlockSpec((1, 16), lambda i: (0, 0)),
            pl.BlockSpec((16, 1), lambda i: (0, 0)),
            pl.BlockSpec((1, 1), lambda i: (0, 0)),
        ],
        out_specs=pl.BlockSpec((BN, out_cols), lambda i: (i, 0)),
        out_shape=jax.ShapeDtypeStruct((N, out_cols), jnp.float32),
    )(xp, agg, w2ax, w2ag, b2a, w2b, b2b, q, p, wh1, bh1, wh2, bh2)


def kernel(x, edge_index, edge_attr, W1a, b1a, W1b, b1b, W2a, b2a, W2b, b2b, Wh1, bh1, Wh2, bh2):
    src = edge_index[0]
    dst = edge_index[1]

    bucket = ((dst >> 6) * 5243) >> 17
    key = (bucket << 22) | lax.iota(jnp.int32, E)
    key_s, dst_s = lax.sort((key, dst), dimension=0, num_keys=1)
    starts = jnp.searchsorted(
        key_s, jnp.arange(NB, dtype=jnp.int32) << 22, side="left").astype(jnp.int32)
    lims = jnp.zeros((80,), jnp.int32)
    lims = lims.at[:NB].set(starts).at[NB].set(E)

    key_p = jnp.pad(key_s, (0, KPAD - E))
    dst_p = jnp.pad(dst_s, (0, KPAD - E))
    attr = edge_attr.reshape(E, 1)

    w1p = jnp.pad(W1a[:9], ((0, 7), (0, 0)))
    w10 = W1a[9:10]
    b1 = b1a.reshape(1, 32)
    b2 = b1b.reshape(1, 32)
    w2ax = jnp.pad(W2a[:9], ((0, 7), (0, 0)))
    w2ag = W2a[9:41]
    b2a_r = b2a.reshape(1, 16)
    b2b_r = b2b.reshape(1, 8)
    q = jnp.zeros((16, 16), jnp.float32).at[0, 0].set(1.0)
    p = jnp.zeros((8, 16), jnp.float32)
    p = p.at[jnp.arange(8), jnp.arange(1, 9)].set(1.0)
    bh1_r = bh1.reshape(1, 16)
    bh2_r = bh2.reshape(1, 1)

    xp = jnp.pad(x, ((0, 0), (0, 7)))
    for layer in range(3):
        g = _gather(xp, src)
        msg = _edge_mlp(g, attr, w1p, w10, b1, W1b, b2)
        agg = _scatter(msg, key_p, dst_p, lims).reshape(NAGG, 32)
        xp = _node_mlp(layer == 2, xp, agg, w2ax, w2ag, b2a_r, W2b, b2b_r,
                       q, p, Wh1, bh1_r, Wh2, bh2_r)
    return xp


# speedup vs baseline: 5.3651x; 1.7971x over previous
import functools

import jax
import jax.numpy as jnp
from jax import lax
from jax.experimental import pallas as pl
from jax.experimental.pallas import tpu as pltpu
from jax.experimental.pallas import tpu_sc as plsc

N = 100000
E = 3200000
NB = 64
NPB = 1600
NAGG = NB * NPB
EPW = E // 32
CHB = 504
CHD = 256
KPAD = E + CHD
PMASK = (1 << 22) - 1
BE = 6400
BN = 4000

_SC_MESH = plsc.VectorSubcoreMesh(core_axis_name="c", subcore_axis_name="s")


def _gather_body(xpad_hbm, src_hbm, g_hbm, idx_v, gbuf, sem):
    c = lax.axis_index("c")
    s = lax.axis_index("s")
    base = (s * 2 + c) * EPW

    nfull = EPW // CHB

    def chunk(k, _):
        off = base + jnp.minimum(k * CHB, EPW - CHB)
        pltpu.sync_copy(src_hbm.at[pl.ds(off, CHB)], idx_v)
        pltpu.async_copy(xpad_hbm.at[idx_v], gbuf, sem).wait()
        pltpu.sync_copy(gbuf, g_hbm.at[pl.ds(off, CHB)])
        return 0

    lax.fori_loop(0, nfull + (1 if EPW % CHB else 0), chunk, 0)


_gather = pl.kernel(
    _gather_body,
    out_type=jax.ShapeDtypeStruct((E, 16), jnp.float32),
    mesh=_SC_MESH,
    compiler_params=pltpu.CompilerParams(use_tc_tiling_on_sc=False),
    scratch_types=[
        pltpu.VMEM((CHB,), jnp.int32),
        pltpu.VMEM((CHB, 16), jnp.float32),
        pltpu.SemaphoreType.DMA,
    ],
)


def _scatter_body(msg_hbm, key_hbm, dst_hbm, lim_hbm, agg_hbm,
                  idx_v, mbuf, dbuf, kbuf, lim_s, acc, sem):
    c = lax.axis_index("c")
    s = lax.axis_index("s")
    tile = s * 2 + c

    pltpu.sync_copy(lim_hbm, lim_s)

    def one_bucket(w):
        def initrow(i, _):
            acc[pl.ds(i * 16, 16)] = jnp.full((16,), -jnp.inf, jnp.float32)
            return 0

        lax.fori_loop(0, (NPB + 1) * 2, initrow, 0)

        lv = lim_s[pl.ds(w, 16)]
        start = lv[0]
        end = lv[1]
        start_al = (start // CHD) * CHD
        nchunks = (end - start_al + CHD - 1) // CHD
        nbase = w * NPB

        def chunk(k, _):
            off = start_al + k * CHD
            pltpu.sync_copy(key_hbm.at[pl.ds(off, CHD)], kbuf)

            def mkidx(vi, _):
                b = vi * 16
                idx_v[pl.ds(b, 16)] = kbuf[pl.ds(b, 16)] & PMASK
                return 0

            lax.fori_loop(0, CHD // 16, mkidx, 0)
            cp = pltpu.async_copy(msg_hbm.at[idx_v], mbuf, sem)
            pltpu.sync_copy(dst_hbm.at[pl.ds(off, CHD)], dbuf)
            cp.wait()
            lo = start - off
            hi = end - off

            def vec16(vi, _):
                base = vi * 16
                lane = lax.iota(jnp.int32, 16) + base
                inb = (lane >= lo) & (lane < hi)
                dv = jnp.where(inb, dbuf[pl.ds(base, 16)] - nbase, NPB)
                for j in range(16):
                    a = dv[j] * 32
                    i = base + j
                    acc[pl.ds(a, 16)] = jnp.maximum(acc[pl.ds(a, 16)], mbuf[i, 0:16])
                    acc[pl.ds(a + 16, 16)] = jnp.maximum(
                        acc[pl.ds(a + 16, 16)], mbuf[i, 16:32])
                return 0

            lax.fori_loop(0, CHD // 16, vec16, 0)
            return 0

        lax.fori_loop(0, nchunks, chunk, 0)
        pltpu.sync_copy(acc.at[pl.ds(0, NPB * 32)],
                        agg_hbm.at[pl.ds(nbase * 32, NPB * 32)])

    one_bucket(tile * 2)
    one_bucket(tile * 2 + 1)


_scatter = pl.kernel(
    _scatter_body,
    out_type=jax.ShapeDtypeStruct((NAGG * 32,), jnp.float32),
    mesh=_SC_MESH,
    compiler_params=pltpu.CompilerParams(use_tc_tiling_on_sc=False),
    scratch_types=[
        pltpu.VMEM((CHD,), jnp.int32),
        pltpu.VMEM((CHD, 32), jnp.float32),
        pltpu.VMEM((CHD,), jnp.int32),
        pltpu.VMEM((CHD,), jnp.int32),
        pltpu.VMEM((80,), jnp.int32),
        pltpu.VMEM(((NPB + 1) * 32,), jnp.float32),
        pltpu.SemaphoreType.DMA,
    ],
)


BEP = BE // 8


def _edge_mlp_body(g_ref, a_ref, w1_ref, s_ref, b1_ref, w2_ref, b2_ref, o_ref):
    h = jnp.maximum(
        g_ref[...] @ w1_ref[...] + a_ref[...] @ s_ref[...] + b1_ref[...], 0.0)
    m = jnp.maximum(h @ w2_ref[...] + b2_ref[...], 0.0)
    o_ref[...] = m.reshape(2 * BEP, 128)


def _edge_mlp(gp, ap, w1bd, sbd, b1t, w2bd, b2t):
    return pl.pallas_call(
        _edge_mlp_body,
        grid=(E // BE,),
        in_specs=[
            pl.BlockSpec((BEP, 128), lambda i: (i, 0)),
            pl.BlockSpec((BEP, 8), lambda i: (i, 0)),
            pl.BlockSpec((128, 256), lambda i: (0, 0)),
            pl.BlockSpec((8, 256), lambda i: (0, 0)),
            pl.BlockSpec((1, 256), lambda i: (0, 0)),
            pl.BlockSpec((256, 256), lambda i: (0, 0)),
            pl.BlockSpec((1, 256), lambda i: (0, 0)),
        ],
        out_specs=pl.BlockSpec((2 * BEP, 128), lambda i: (i, 0)),
        out_shape=jax.ShapeDtypeStruct((E // 4, 128), jnp.float32),
    )(gp, ap, w1bd, sbd, b1t, w2bd, b2t)


def _node_mlp_body(final, xp_ref, agg_ref, w2ax_ref, w2ag_ref, b2a_ref,
                   w2b_ref, b2b_ref, q_ref, p_ref, wh1_ref, bh1_ref,
                   wh2_ref, bh2_ref, o_ref):
    xp = xp_ref[...]
    agg = agg_ref[...]
    agg = jnp.where(agg == -jnp.inf, 0.0, agg)
    h = jnp.maximum(xp @ w2ax_ref[...] + agg @ w2ag_ref[...] + b2a_ref[...], 0.0)
    comb = jnp.maximum(h @ w2b_ref[...] + b2b_ref[...], 0.0)
    if final:
        h2 = jnp.maximum(comb @ wh1_ref[...] + bh1_ref[...], 0.0)
        z = h2 @ wh2_ref[...] + bh2_ref[...]
        o_ref[...] = 1.0 / (1.0 + jnp.exp(-z))
    else:
        o_ref[...] = xp @ q_ref[...] + comb @ p_ref[...]


def _node_mlp(final, xp, agg, w2ax, w2ag, b2a, w2b, b2b, q, p, wh1, bh1, wh2, bh2):
    out_cols = 1 if final else 16
    return pl.pallas_call(
        functools.partial(_node_mlp_body, final),
        grid=(N // BN,),
        in_specs=[
            pl.BlockSpec((BN, 16), lambda i: (i, 0)),
            pl.BlockSpec((BN, 32), lambda i: (i, 0)),
            pl.BlockSpec((16, 16), lambda i: (0, 0)),
            pl.BlockSpec((32, 16), lambda i: (0, 0)),
            pl.BlockSpec((1, 16), lambda i: (0, 0)),
            pl.BlockSpec((16, 8), lambda i: (0, 0)),
            pl.BlockSpec((1, 8), lambda i: (0, 0)),
            pl.BlockSpec((16, 16), lambda i: (0, 0)),
            pl.BlockSpec((8, 16), lambda i: (0, 0)),
            pl.BlockSpec((8, 16), lambda i: (0, 0)),
            pl.BlockSpec((1, 16), lambda i: (0, 0)),
            pl.BlockSpec((16, 1), lambda i: (0, 0)),
            pl.BlockSpec((1, 1), lambda i: (0, 0)),
        ],
        out_specs=pl.BlockSpec((BN, out_cols), lambda i: (i, 0)),
        out_shape=jax.ShapeDtypeStruct((N, out_cols), jnp.float32),
    )(xp, agg, w2ax, w2ag, b2a, w2b, b2b, q, p, wh1, bh1, wh2, bh2)


def kernel(x, edge_index, edge_attr, W1a, b1a, W1b, b1b, W2a, b2a, W2b, b2b, Wh1, bh1, Wh2, bh2):
    src = edge_index[0]
    dst = edge_index[1]

    bucket = ((dst >> 6) * 5243) >> 17
    key = (bucket << 22) | lax.iota(jnp.int32, E)
    key_s, dst_s = lax.sort((key, dst), dimension=0, num_keys=1)
    starts = jnp.searchsorted(
        key_s, jnp.arange(NB, dtype=jnp.int32) << 22, side="left").astype(jnp.int32)
    lims = jnp.zeros((80,), jnp.int32)
    lims = lims.at[:NB].set(starts).at[NB].set(E)

    key_p = jnp.pad(key_s, (0, KPAD - E))
    dst_p = jnp.pad(dst_s, (0, KPAD - E))
    ap = edge_attr.reshape(E // 8, 8)

    eye8 = jnp.eye(8, dtype=jnp.float32)
    w1p = jnp.pad(W1a[:9], ((0, 7), (0, 0)))
    w1bd = jnp.kron(eye8, w1p)
    sbd = jnp.kron(eye8, W1a[9:10])
    b1t = jnp.tile(b1a, (8,)).reshape(1, 256)
    w2bd = jnp.kron(eye8, W1b)
    b2t = jnp.tile(b1b, (8,)).reshape(1, 256)
    w2ax = jnp.pad(W2a[:9], ((0, 7), (0, 0)))
    w2ag = W2a[9:41]
    b2a_r = b2a.reshape(1, 16)
    b2b_r = b2b.reshape(1, 8)
    q = jnp.zeros((16, 16), jnp.float32).at[0, 0].set(1.0)
    p = jnp.zeros((8, 16), jnp.float32)
    p = p.at[jnp.arange(8), jnp.arange(1, 9)].set(1.0)
    bh1_r = bh1.reshape(1, 16)
    bh2_r = bh2.reshape(1, 1)

    xp = jnp.pad(x, ((0, 0), (0, 7)))
    for layer in range(3):
        g = _gather(xp, src)
        gp = g.reshape(E // 8, 128)
        msgp = _edge_mlp(gp, ap, w1bd, sbd, b1t, w2bd, b2t)
        msg = msgp.reshape(E, 32)
        agg = _scatter(msg, key_p, dst_p, lims).reshape(NAGG, 32)
        xp = _node_mlp(layer == 2, xp, agg, w2ax, w2ag, b2a_r, W2b, b2b_r,
                       q, p, Wh1, bh1_r, Wh2, bh2_r)
    return xp


# speedup vs baseline: 5.5655x; 1.0374x over previous
import functools

import jax
import jax.numpy as jnp
from jax import lax
from jax.experimental import pallas as pl
from jax.experimental.pallas import tpu as pltpu
from jax.experimental.pallas import tpu_sc as plsc

N = 100000
E = 3200000
NB = 64
NPB = 1600
NAGG = NB * NPB
EPW = E // 32
CHB = 504
CHD = 256
KPAD = E + CHD
PMASK = (1 << 22) - 1
BE = 6400
BN = 4000

_SC_MESH = plsc.VectorSubcoreMesh(core_axis_name="c", subcore_axis_name="s")


def _gather_body(xpad_hbm, src_hbm, g_hbm, idx_v, gbuf, sem):
    c = lax.axis_index("c")
    s = lax.axis_index("s")
    base = (s * 2 + c) * EPW

    nfull = EPW // CHB

    def chunk(k, _):
        off = base + jnp.minimum(k * CHB, EPW - CHB)
        pltpu.sync_copy(src_hbm.at[pl.ds(off, CHB)], idx_v)
        pltpu.async_copy(xpad_hbm.at[idx_v], gbuf, sem).wait()
        pltpu.sync_copy(gbuf, g_hbm.at[pl.ds(off, CHB)])
        return 0

    lax.fori_loop(0, nfull + (1 if EPW % CHB else 0), chunk, 0)


_gather = pl.kernel(
    _gather_body,
    out_type=jax.ShapeDtypeStruct((E, 16), jnp.float32),
    mesh=_SC_MESH,
    compiler_params=pltpu.CompilerParams(use_tc_tiling_on_sc=False),
    scratch_types=[
        pltpu.VMEM((CHB,), jnp.int32),
        pltpu.VMEM((CHB, 16), jnp.float32),
        pltpu.SemaphoreType.DMA,
    ],
)


def _scatter_body(msg_hbm, key_hbm, dst_hbm, lim_hbm, agg_hbm,
                  idx_v, mbuf, dbuf, kbuf, lim_s, acc, sem, sem2):
    c = lax.axis_index("c")
    s = lax.axis_index("s")
    tile = s * 2 + c

    pltpu.sync_copy(lim_hbm, lim_s)

    def one_bucket(w):
        def initrow(i, _):
            acc[pl.ds(i * 16, 16)] = jnp.full((16,), -jnp.inf, jnp.float32)
            return 0

        lax.fori_loop(0, (NPB + 1) * 2, initrow, 0)

        lv = lim_s[pl.ds(w, 16)]
        start = lv[0]
        end = lv[1]
        start_al = (start // CHD) * CHD
        nchunks = (end - start_al + CHD - 1) // CHD
        nbase = w * NPB

        def chunk(k, _):
            off = start_al + k * CHD
            pltpu.sync_copy(key_hbm.at[pl.ds(off, CHD)], kbuf)

            def mkidx(vi, _):
                b = vi * 16
                idx_v[pl.ds(b, 16)] = kbuf[pl.ds(b, 16)] & PMASK
                return 0

            lax.fori_loop(0, CHD // 16, mkidx, 0)
            cp = pltpu.async_copy(msg_hbm.at[idx_v], mbuf, sem)
            cpd = pltpu.async_copy(dst_hbm.at[idx_v], dbuf, sem2)
            cp.wait()
            cpd.wait()
            lo = start - off
            hi = end - off

            def vec16(vi, _):
                base = vi * 16
                lane = lax.iota(jnp.int32, 16) + base
                inb = (lane >= lo) & (lane < hi)
                dv = jnp.where(inb, dbuf[pl.ds(base, 16)] - nbase, NPB)
                for j in range(16):
                    a = dv[j] * 32
                    i = base + j
                    acc[pl.ds(a, 16)] = jnp.maximum(acc[pl.ds(a, 16)], mbuf[i, 0:16])
                    acc[pl.ds(a + 16, 16)] = jnp.maximum(
                        acc[pl.ds(a + 16, 16)], mbuf[i, 16:32])
                return 0

            lax.fori_loop(0, CHD // 16, vec16, 0)
            return 0

        lax.fori_loop(0, nchunks, chunk, 0)
        pltpu.sync_copy(acc.at[pl.ds(0, NPB * 32)],
                        agg_hbm.at[pl.ds(nbase * 32, NPB * 32)])

    one_bucket(tile * 2)
    one_bucket(tile * 2 + 1)


_scatter = pl.kernel(
    _scatter_body,
    out_type=jax.ShapeDtypeStruct((NAGG * 32,), jnp.float32),
    mesh=_SC_MESH,
    compiler_params=pltpu.CompilerParams(use_tc_tiling_on_sc=False),
    scratch_types=[
        pltpu.VMEM((CHD,), jnp.int32),
        pltpu.VMEM((CHD, 32), jnp.float32),
        pltpu.VMEM((CHD,), jnp.int32),
        pltpu.VMEM((CHD,), jnp.int32),
        pltpu.VMEM((80,), jnp.int32),
        pltpu.VMEM(((NPB + 1) * 32,), jnp.float32),
        pltpu.SemaphoreType.DMA,
        pltpu.SemaphoreType.DMA,
    ],
)


BEP = BE // 8


def _edge_mlp_body(g_ref, a_ref, w1_ref, s_ref, b1_ref, w2_ref, b2_ref, o_ref):
    h = jnp.maximum(
        g_ref[...] @ w1_ref[...] + a_ref[...] @ s_ref[...] + b1_ref[...], 0.0)
    m = jnp.maximum(h @ w2_ref[...] + b2_ref[...], 0.0)
    o_ref[...] = m.reshape(2 * BEP, 128)


def _edge_mlp(gp, ap, w1bd, sbd, b1t, w2bd, b2t):
    return pl.pallas_call(
        _edge_mlp_body,
        grid=(E // BE,),
        in_specs=[
            pl.BlockSpec((BEP, 128), lambda i: (i, 0)),
            pl.BlockSpec((BEP, 8), lambda i: (i, 0)),
            pl.BlockSpec((128, 256), lambda i: (0, 0)),
            pl.BlockSpec((8, 256), lambda i: (0, 0)),
            pl.BlockSpec((1, 256), lambda i: (0, 0)),
            pl.BlockSpec((256, 256), lambda i: (0, 0)),
            pl.BlockSpec((1, 256), lambda i: (0, 0)),
        ],
        out_specs=pl.BlockSpec((2 * BEP, 128), lambda i: (i, 0)),
        out_shape=jax.ShapeDtypeStruct((E // 4, 128), jnp.float32),
    )(gp, ap, w1bd, sbd, b1t, w2bd, b2t)


def _node_mlp_body(final, xp_ref, agg_ref, w2ax_ref, w2ag_ref, b2a_ref,
                   w2b_ref, b2b_ref, q_ref, p_ref, wh1_ref, bh1_ref,
                   wh2_ref, bh2_ref, o_ref):
    xp = xp_ref[...]
    agg = agg_ref[...]
    agg = jnp.where(agg == -jnp.inf, 0.0, agg)
    h = jnp.maximum(xp @ w2ax_ref[...] + agg @ w2ag_ref[...] + b2a_ref[...], 0.0)
    comb = jnp.maximum(h @ w2b_ref[...] + b2b_ref[...], 0.0)
    if final:
        h2 = jnp.maximum(comb @ wh1_ref[...] + bh1_ref[...], 0.0)
        z = h2 @ wh2_ref[...] + bh2_ref[...]
        o_ref[...] = 1.0 / (1.0 + jnp.exp(-z))
    else:
        o_ref[...] = xp @ q_ref[...] + comb @ p_ref[...]


def _node_mlp(final, xp, agg, w2ax, w2ag, b2a, w2b, b2b, q, p, wh1, bh1, wh2, bh2):
    out_cols = 1 if final else 16
    return pl.pallas_call(
        functools.partial(_node_mlp_body, final),
        grid=(N // BN,),
        in_specs=[
            pl.BlockSpec((BN, 16), lambda i: (i, 0)),
            pl.BlockSpec((BN, 32), lambda i: (i, 0)),
            pl.BlockSpec((16, 16), lambda i: (0, 0)),
            pl.BlockSpec((32, 16), lambda i: (0, 0)),
            pl.BlockSpec((1, 16), lambda i: (0, 0)),
            pl.BlockSpec((16, 8), lambda i: (0, 0)),
            pl.BlockSpec((1, 8), lambda i: (0, 0)),
            pl.BlockSpec((16, 16), lambda i: (0, 0)),
            pl.BlockSpec((8, 16), lambda i: (0, 0)),
            pl.BlockSpec((8, 16), lambda i: (0, 0)),
            pl.BlockSpec((1, 16), lambda i: (0, 0)),
            pl.BlockSpec((16, 1), lambda i: (0, 0)),
            pl.BlockSpec((1, 1), lambda i: (0, 0)),
        ],
        out_specs=pl.BlockSpec((BN, out_cols), lambda i: (i, 0)),
        out_shape=jax.ShapeDtypeStruct((N, out_cols), jnp.float32),
    )(xp, agg, w2ax, w2ag, b2a, w2b, b2b, q, p, wh1, bh1, wh2, bh2)


def kernel(x, edge_index, edge_attr, W1a, b1a, W1b, b1b, W2a, b2a, W2b, b2b, Wh1, bh1, Wh2, bh2):
    src = edge_index[0]
    dst = edge_index[1]

    bucket = ((dst >> 6) * 5243) >> 17
    key = (bucket << 22) | lax.iota(jnp.int32, E)
    key_s = lax.sort(key, dimension=0)
    starts = jnp.searchsorted(
        key_s, jnp.arange(NB, dtype=jnp.int32) << 22, side="left").astype(jnp.int32)
    lims = jnp.zeros((80,), jnp.int32)
    lims = lims.at[:NB].set(starts).at[NB].set(E)

    key_p = jnp.pad(key_s, (0, KPAD - E))
    ap = edge_attr.reshape(E // 8, 8)

    eye8 = jnp.eye(8, dtype=jnp.float32)
    w1p = jnp.pad(W1a[:9], ((0, 7), (0, 0)))
    w1bd = jnp.kron(eye8, w1p)
    sbd = jnp.kron(eye8, W1a[9:10])
    b1t = jnp.tile(b1a, (8,)).reshape(1, 256)
    w2bd = jnp.kron(eye8, W1b)
    b2t = jnp.tile(b1b, (8,)).reshape(1, 256)
    w2ax = jnp.pad(W2a[:9], ((0, 7), (0, 0)))
    w2ag = W2a[9:41]
    b2a_r = b2a.reshape(1, 16)
    b2b_r = b2b.reshape(1, 8)
    q = jnp.zeros((16, 16), jnp.float32).at[0, 0].set(1.0)
    p = jnp.zeros((8, 16), jnp.float32)
    p = p.at[jnp.arange(8), jnp.arange(1, 9)].set(1.0)
    bh1_r = bh1.reshape(1, 16)
    bh2_r = bh2.reshape(1, 1)

    xp = jnp.pad(x, ((0, 0), (0, 7)))
    for layer in range(3):
        g = _gather(xp, src)
        gp = g.reshape(E // 8, 128)
        msgp = _edge_mlp(gp, ap, w1bd, sbd, b1t, w2bd, b2t)
        msg = msgp.reshape(E, 32)
        agg = _scatter(msg, key_p, dst, lims).reshape(NAGG, 32)
        xp = _node_mlp(layer == 2, xp, agg, w2ax, w2ag, b2a_r, W2b, b2b_r,
                       q, p, Wh1, bh1_r, Wh2, bh2_r)
    return xp
